# baseline (speedup 1.0000x reference)
import jax
import jax.numpy as jnp
from jax import lax
from jax.experimental import pallas as pl
from jax.experimental.pallas import tpu as pltpu

B = 16
N1 = 1024
M1 = 512
M2 = 128
TI = 8


def _fps_kernel(px_ref, py_ref, pz_ref, s1x_ref, s1y_ref, s1z_ref,
                s2x_ref, s2y_ref, s2z_ref):
    def run(px, py, pz, n, m, ox_ref, oy_ref, oz_ref):
        rows = lax.broadcasted_iota(jnp.int32, (n, B), 0)

        def body(i, carry):
            far, mind = carry
            sel = rows == far
            cx = jnp.max(jnp.where(sel, px, -1e30), axis=0, keepdims=True)
            cy = jnp.max(jnp.where(sel, py, -1e30), axis=0, keepdims=True)
            cz = jnp.max(jnp.where(sel, pz, -1e30), axis=0, keepdims=True)
            ox_ref[pl.ds(i, 1), :] = cx
            oy_ref[pl.ds(i, 1), :] = cy
            oz_ref[pl.ds(i, 1), :] = cz
            d = (px - cx) ** 2 + (py - cy) ** 2 + (pz - cz) ** 2
            mind = jnp.minimum(mind, d)
            mx = jnp.max(mind, axis=0, keepdims=True)
            far = jnp.min(jnp.where(mind == mx, rows, n), axis=0, keepdims=True)
            return far, mind

        far0 = jnp.zeros((1, B), jnp.int32)
        mind0 = jnp.full((n, B), 1e10, jnp.float32)
        lax.fori_loop(0, m, body, (far0, mind0))

    run(px_ref[:, :], py_ref[:, :], pz_ref[:, :], N1, M1,
        s1x_ref, s1y_ref, s1z_ref)
    run(s1x_ref[:, :], s1y_ref[:, :], s1z_ref[:, :], M1, M2,
        s2x_ref, s2y_ref, s2z_ref)


def _sa_tail(u, poscols, psel, vv, w2, b2, r2, out_ref):
    w2b = w2.astype(jnp.bfloat16)
    for t in range(TI):
        a = jnp.maximum(u + vv[t:t + 1, :], 0.0).astype(jnp.bfloat16)
        o = jnp.dot(a, w2b, preferred_element_type=jnp.float32) + b2
        d2 = ((poscols[:, 0:1] - psel[t:t + 1, 0:1]) ** 2 +
              (poscols[:, 1:2] - psel[t:t + 1, 1:2]) ** 2 +
              (poscols[:, 2:3] - psel[t:t + 1, 2:3]) ** 2)
        pen = jnp.where(d2 <= r2, 0.0, -1e10)
        out_ref[0, pl.ds(t, 1), :] = jnp.max(o + pen, axis=0, keepdims=True)


def _sa1_kernel(x_ref, psel_ref, wu_ref, bu_ref, wv_ref, w2_ref, b2_ref,
                out_ref, u_ref):
    @pl.when(pl.program_id(1) == 0)
    def _():
        u_ref[:, :] = jnp.dot(x_ref[0], wu_ref[:, :],
                              preferred_element_type=jnp.float32)

    psel = psel_ref[0]
    vv = bu_ref[:, :] - jnp.dot(psel, wv_ref[:, :],
                                preferred_element_type=jnp.float32)
    _sa_tail(u_ref[:, :], x_ref[0][:, 0:3], psel, vv, w2_ref[:, :],
             b2_ref[:, :], 0.2 * 0.2, out_ref)


def _sa2_kernel(h1_ref, p1_ref, psel_ref, wa_ref, bu_ref, wv_ref, w2_ref,
                b2_ref, out_ref, u_ref):
    @pl.when(pl.program_id(1) == 0)
    def _():
        u_ref[:, :] = (
            jnp.dot(h1_ref[0], wa_ref[:, :], preferred_element_type=jnp.float32)
            + jnp.dot(p1_ref[0], wv_ref[:, :],
                      preferred_element_type=jnp.float32))

    psel = psel_ref[0]
    vv = bu_ref[:, :] - jnp.dot(psel, wv_ref[:, :],
                                preferred_element_type=jnp.float32)
    _sa_tail(u_ref[:, :], p1_ref[0], psel, vv, w2_ref[:, :], b2_ref[:, :],
             0.4 * 0.4, out_ref)


def _head_kernel(x3_ref, w31_ref, b31_ref, w32_ref, b32_ref, wh1_ref, bh1_ref,
                 wh2_ref, bh2_ref, out_ref):
    h = jnp.maximum(jnp.dot(x3_ref[:, :], w31_ref[:, :],
                            preferred_element_type=jnp.float32)
                    + b31_ref[:, :], 0.0)
    g = jnp.dot(h, w32_ref[:, :], preferred_element_type=jnp.float32) \
        + b32_ref[:, :]
    gm = jnp.concatenate(
        [jnp.max(g[b * M2:(b + 1) * M2, :], axis=0, keepdims=True)
         for b in range(B)], axis=0)
    h2 = jnp.maximum(jnp.dot(gm, wh1_ref[:, :],
                             preferred_element_type=jnp.float32)
                     + bh1_ref[:, :], 0.0)
    out_ref[:, :] = jnp.dot(h2, wh2_ref[:, :],
                            preferred_element_type=jnp.float32) + bh2_ref[:, :]


def kernel(x, W1_1, b1_1, W1_2, b1_2, W2_1, b2_1, W2_2, b2_2, W3_1, b3_1,
           W3_2, b3_2, Wh1, bh1, Wh2, bh2):
    pos = x[..., :3]
    px = jnp.transpose(pos[:, :, 0])
    py = jnp.transpose(pos[:, :, 1])
    pz = jnp.transpose(pos[:, :, 2])

    f32 = jnp.float32
    s1x, s1y, s1z, s2x, s2y, s2z = pl.pallas_call(
        _fps_kernel,
        grid=(1,),
        in_specs=[pl.BlockSpec((N1, B), lambda i: (0, 0))] * 3,
        out_specs=[pl.BlockSpec((M1, B), lambda i: (0, 0))] * 3
        + [pl.BlockSpec((M2, B), lambda i: (0, 0))] * 3,
        out_shape=[jax.ShapeDtypeStruct((M1, B), f32)] * 3
        + [jax.ShapeDtypeStruct((M2, B), f32)] * 3,
    )(px, py, pz)

    pos1 = jnp.stack([jnp.transpose(s1x), jnp.transpose(s1y),
                      jnp.transpose(s1z)], axis=-1)
    pos2 = jnp.stack([jnp.transpose(s2x), jnp.transpose(s2y),
                      jnp.transpose(s2z)], axis=-1)

    wu1 = jnp.concatenate([W1_1[3:6], W1_1[0:3]], axis=0)
    wv1 = W1_1[3:6]
    nI1 = M1 // TI

    h1 = pl.pallas_call(
        _sa1_kernel,
        grid=(B, nI1),
        in_specs=[
            pl.BlockSpec((1, N1, 6), lambda b, i: (b, 0, 0)),
            pl.BlockSpec((1, TI, 3), lambda b, i: (b, i, 0)),
            pl.BlockSpec((6, 32), lambda b, i: (0, 0)),
            pl.BlockSpec((1, 32), lambda b, i: (0, 0)),
            pl.BlockSpec((3, 32), lambda b, i: (0, 0)),
            pl.BlockSpec((32, 64), lambda b, i: (0, 0)),
            pl.BlockSpec((1, 64), lambda b, i: (0, 0)),
        ],
        out_specs=pl.BlockSpec((1, TI, 64), lambda b, i: (b, i, 0)),
        out_shape=jax.ShapeDtypeStruct((B, M1, 64), f32),
        scratch_shapes=[pltpu.VMEM((N1, 32), f32)],
    )(x, pos1, wu1, b1_1.reshape(1, 32), wv1, W1_2, b1_2.reshape(1, 64))

    wa2 = W2_1[0:64]
    wv2 = W2_1[64:67]
    nI2 = M2 // TI

    h2 = pl.pallas_call(
        _sa2_kernel,
        grid=(B, nI2),
        in_specs=[
            pl.BlockSpec((1, M1, 64), lambda b, i: (b, 0, 0)),
            pl.BlockSpec((1, M1, 3), lambda b, i: (b, 0, 0)),
            pl.BlockSpec((1, TI, 3), lambda b, i: (b, i, 0)),
            pl.BlockSpec((64, 64), lambda b, i: (0, 0)),
            pl.BlockSpec((1, 64), lambda b, i: (0, 0)),
            pl.BlockSpec((3, 64), lambda b, i: (0, 0)),
            pl.BlockSpec((64, 128), lambda b, i: (0, 0)),
            pl.BlockSpec((1, 128), lambda b, i: (0, 0)),
        ],
        out_specs=pl.BlockSpec((1, TI, 128), lambda b, i: (b, i, 0)),
        out_shape=jax.ShapeDtypeStruct((B, M2, 128), f32),
        scratch_shapes=[pltpu.VMEM((M1, 64), f32)],
    )(h1, pos1, pos2, wa2, b2_1.reshape(1, 64), wv2, W2_2,
      b2_2.reshape(1, 128))

    x3 = jnp.concatenate([h2, pos2], axis=-1).reshape(B * M2, 131)
    out = pl.pallas_call(
        _head_kernel,
        grid=(1,),
        in_specs=[
            pl.BlockSpec((B * M2, 131), lambda i: (0, 0)),
            pl.BlockSpec((131, 256), lambda i: (0, 0)),
            pl.BlockSpec((1, 256), lambda i: (0, 0)),
            pl.BlockSpec((256, 512), lambda i: (0, 0)),
            pl.BlockSpec((1, 512), lambda i: (0, 0)),
            pl.BlockSpec((512, 256), lambda i: (0, 0)),
            pl.BlockSpec((1, 256), lambda i: (0, 0)),
            pl.BlockSpec((256, 40), lambda i: (0, 0)),
            pl.BlockSpec((1, 40), lambda i: (0, 0)),
        ],
        out_specs=pl.BlockSpec((B, 40), lambda i: (0, 0)),
        out_shape=jax.ShapeDtypeStruct((B, 40), f32),
    )(x3, W3_1, b3_1.reshape(1, 256), W3_2, b3_2.reshape(1, 512),
      Wh1, bh1.reshape(1, 256), Wh2, bh2.reshape(1, 40))
    return out


# speedup vs baseline: 2.5022x; 2.5022x over previous
import jax
import jax.numpy as jnp
from jax import lax
from jax.experimental import pallas as pl
from jax.experimental.pallas import tpu as pltpu

B = 16
N1 = 1024
M1 = 512
M2 = 128
TI = 8


def _fps_kernel(px_ref, py_ref, pz_ref, s1x_ref, s1y_ref, s1z_ref,
                s2x_ref, s2y_ref, s2z_ref):
    def run(px, py, pz, n, m, ox_ref, oy_ref, oz_ref):
        rows = lax.broadcasted_iota(jnp.int32, (n, B), 0)

        def body(i, carry):
            far, mind = carry
            sel = rows == far
            cx = jnp.max(jnp.where(sel, px, -1e30), axis=0, keepdims=True)
            cy = jnp.max(jnp.where(sel, py, -1e30), axis=0, keepdims=True)
            cz = jnp.max(jnp.where(sel, pz, -1e30), axis=0, keepdims=True)
            ox_ref[pl.ds(i, 1), :] = cx
            oy_ref[pl.ds(i, 1), :] = cy
            oz_ref[pl.ds(i, 1), :] = cz
            d = (px - cx) ** 2 + (py - cy) ** 2 + (pz - cz) ** 2
            mind = jnp.minimum(mind, d)
            mx = jnp.max(mind, axis=0, keepdims=True)
            far = jnp.min(jnp.where(mind == mx, rows, n), axis=0, keepdims=True)
            return far, mind

        far0 = jnp.zeros((1, B), jnp.int32)
        mind0 = jnp.full((n, B), 1e10, jnp.float32)
        lax.fori_loop(0, m, body, (far0, mind0))

    run(px_ref[:, :], py_ref[:, :], pz_ref[:, :], N1, M1,
        s1x_ref, s1y_ref, s1z_ref)
    run(s1x_ref[:, :], s1y_ref[:, :], s1z_ref[:, :], M1, M2,
        s2x_ref, s2y_ref, s2z_ref)


def _sa_tail(utile_ref, prow, psel, pselflat, bvflat_ref, wvblk_ref, w2blk_ref,
             penblk_ref, b2tile_ref, r2, out_ref):
    vvflat = bvflat_ref[:, :] - jnp.dot(pselflat, wvblk_ref[:, :],
                                        preferred_element_type=jnp.float32)
    a = jnp.maximum(utile_ref[:, :] + vvflat, 0.0).astype(jnp.bfloat16)
    o = jnp.dot(a, w2blk_ref[:, :], preferred_element_type=jnp.float32)
    d2 = ((psel[:, 0:1] - prow[0:1, :]) ** 2 +
          (psel[:, 1:2] - prow[1:2, :]) ** 2 +
          (psel[:, 2:3] - prow[2:3, :]) ** 2)
    penval = jnp.where(jnp.transpose(d2) <= r2, 0.0, 1.0)
    o = o + jnp.dot(penval, penblk_ref[:, :],
                    preferred_element_type=jnp.float32)
    out_ref[0, 0, :, :] = jnp.max(o, axis=0, keepdims=True) + b2tile_ref[:, :]


def _sa1_kernel(x_ref, prow_ref, psel_ref, pself_ref, wu_ref, bvflat_ref,
                wvblk_ref, w2blk_ref, penblk_ref, b2tile_ref, out_ref,
                utile_ref):
    @pl.when(pl.program_id(1) == 0)
    def _():
        u = jnp.dot(x_ref[0], wu_ref[:, :], preferred_element_type=jnp.float32)
        utile_ref[:, :] = jnp.concatenate([u] * TI, axis=1)

    _sa_tail(utile_ref, prow_ref[0], psel_ref[0], pself_ref[0, 0],
             bvflat_ref, wvblk_ref, w2blk_ref, penblk_ref, b2tile_ref,
             0.2 * 0.2, out_ref)


def _sa2_kernel(h1_ref, p1_ref, prow_ref, psel_ref, pself_ref, wa_ref,
                wv_ref, bvflat_ref, wvblk_ref, w2blk_ref, penblk_ref,
                b2tile_ref, out_ref, utile_ref):
    @pl.when(pl.program_id(1) == 0)
    def _():
        u = (jnp.dot(h1_ref[0], wa_ref[:, :],
                     preferred_element_type=jnp.float32)
             + jnp.dot(p1_ref[0], wv_ref[:, :],
                       preferred_element_type=jnp.float32))
        utile_ref[:, :] = jnp.concatenate([u] * TI, axis=1)

    _sa_tail(utile_ref, prow_ref[0], psel_ref[0], pself_ref[0, 0],
             bvflat_ref, wvblk_ref, w2blk_ref, penblk_ref, b2tile_ref,
             0.4 * 0.4, out_ref)


def _head_kernel(x3_ref, w31_ref, b31_ref, w32_ref, b32_ref, wh1_ref, bh1_ref,
                 wh2_ref, bh2_ref, out_ref):
    h = jnp.maximum(jnp.dot(x3_ref[:, :], w31_ref[:, :],
                            preferred_element_type=jnp.float32)
                    + b31_ref[:, :], 0.0)
    g = jnp.dot(h, w32_ref[:, :], preferred_element_type=jnp.float32) \
        + b32_ref[:, :]
    gm = jnp.concatenate(
        [jnp.max(g[b * M2:(b + 1) * M2, :], axis=0, keepdims=True)
         for b in range(B)], axis=0)
    h2 = jnp.maximum(jnp.dot(gm, wh1_ref[:, :],
                             preferred_element_type=jnp.float32)
                     + bh1_ref[:, :], 0.0)
    out_ref[:, :] = jnp.dot(h2, wh2_ref[:, :],
                            preferred_element_type=jnp.float32) + bh2_ref[:, :]


def kernel(x, W1_1, b1_1, W1_2, b1_2, W2_1, b2_1, W2_2, b2_2, W3_1, b3_1,
           W3_2, b3_2, Wh1, bh1, Wh2, bh2):
    pos = x[..., :3]
    px = jnp.transpose(pos[:, :, 0])
    py = jnp.transpose(pos[:, :, 1])
    pz = jnp.transpose(pos[:, :, 2])

    f32 = jnp.float32
    s1x, s1y, s1z, s2x, s2y, s2z = pl.pallas_call(
        _fps_kernel,
        grid=(1,),
        in_specs=[pl.BlockSpec((N1, B), lambda i: (0, 0))] * 3,
        out_specs=[pl.BlockSpec((M1, B), lambda i: (0, 0))] * 3
        + [pl.BlockSpec((M2, B), lambda i: (0, 0))] * 3,
        out_shape=[jax.ShapeDtypeStruct((M1, B), f32)] * 3
        + [jax.ShapeDtypeStruct((M2, B), f32)] * 3,
    )(px, py, pz)

    pos1 = jnp.stack([jnp.transpose(s1x), jnp.transpose(s1y),
                      jnp.transpose(s1z)], axis=-1)
    pos2 = jnp.stack([jnp.transpose(s2x), jnp.transpose(s2y),
                      jnp.transpose(s2z)], axis=-1)

    posT = jnp.transpose(pos, (0, 2, 1))
    pos1T = jnp.transpose(pos1, (0, 2, 1))

    wu1 = jnp.concatenate([W1_1[3:6], W1_1[0:3]], axis=0)
    wv1 = W1_1[3:6]
    nI1 = M1 // TI
    eye = jnp.eye(TI, dtype=f32)
    wvblk1 = jnp.kron(eye, wv1)
    w2blk1 = jnp.kron(eye, W1_2).astype(jnp.bfloat16)
    penblk1 = jnp.kron(eye, jnp.full((1, 64), -1e10, f32))
    bvflat1 = jnp.tile(b1_1.reshape(1, 32), (1, TI))
    b2tile1 = jnp.tile(b1_2.reshape(1, 64), (1, TI))
    pos1flat = pos1.reshape(B, nI1, 1, 3 * TI)

    h1 = pl.pallas_call(
        _sa1_kernel,
        grid=(B, nI1),
        in_specs=[
            pl.BlockSpec((1, N1, 6), lambda b, i: (b, 0, 0)),
            pl.BlockSpec((1, 3, N1), lambda b, i: (b, 0, 0)),
            pl.BlockSpec((1, TI, 3), lambda b, i: (b, i, 0)),
            pl.BlockSpec((1, 1, 1, 3 * TI), lambda b, i: (b, i, 0, 0)),
            pl.BlockSpec((6, 32), lambda b, i: (0, 0)),
            pl.BlockSpec((1, 32 * TI), lambda b, i: (0, 0)),
            pl.BlockSpec((3 * TI, 32 * TI), lambda b, i: (0, 0)),
            pl.BlockSpec((32 * TI, 64 * TI), lambda b, i: (0, 0)),
            pl.BlockSpec((TI, 64 * TI), lambda b, i: (0, 0)),
            pl.BlockSpec((1, 64 * TI), lambda b, i: (0, 0)),
        ],
        out_specs=pl.BlockSpec((1, 1, 1, 64 * TI), lambda b, i: (b, i, 0, 0)),
        out_shape=jax.ShapeDtypeStruct((B, nI1, 1, 64 * TI), f32),
        scratch_shapes=[pltpu.VMEM((N1, 32 * TI), f32)],
    )(x, posT, pos1, pos1flat, wu1, bvflat1, wvblk1, w2blk1, penblk1, b2tile1)
    h1 = h1.reshape(B, M1, 64)

    wa2 = W2_1[0:64]
    wv2 = W2_1[64:67]
    nI2 = M2 // TI
    wvblk2 = jnp.kron(eye, wv2)
    w2blk2 = jnp.kron(eye, W2_2).astype(jnp.bfloat16)
    penblk2 = jnp.kron(eye, jnp.full((1, 128), -1e10, f32))
    bvflat2 = jnp.tile(b2_1.reshape(1, 64), (1, TI))
    b2tile2 = jnp.tile(b2_2.reshape(1, 128), (1, TI))
    pos2flat = pos2.reshape(B, nI2, 1, 3 * TI)

    h2 = pl.pallas_call(
        _sa2_kernel,
        grid=(B, nI2),
        in_specs=[
            pl.BlockSpec((1, M1, 64), lambda b, i: (b, 0, 0)),
            pl.BlockSpec((1, M1, 3), lambda b, i: (b, 0, 0)),
            pl.BlockSpec((1, 3, M1), lambda b, i: (b, 0, 0)),
            pl.BlockSpec((1, TI, 3), lambda b, i: (b, i, 0)),
            pl.BlockSpec((1, 1, 1, 3 * TI), lambda b, i: (b, i, 0, 0)),
            pl.BlockSpec((64, 64), lambda b, i: (0, 0)),
            pl.BlockSpec((3, 64), lambda b, i: (0, 0)),
            pl.BlockSpec((1, 64 * TI), lambda b, i: (0, 0)),
            pl.BlockSpec((3 * TI, 64 * TI), lambda b, i: (0, 0)),
            pl.BlockSpec((64 * TI, 128 * TI), lambda b, i: (0, 0)),
            pl.BlockSpec((TI, 128 * TI), lambda b, i: (0, 0)),
            pl.BlockSpec((1, 128 * TI), lambda b, i: (0, 0)),
        ],
        out_specs=pl.BlockSpec((1, 1, 1, 128 * TI), lambda b, i: (b, i, 0, 0)),
        out_shape=jax.ShapeDtypeStruct((B, nI2, 1, 128 * TI), f32),
        scratch_shapes=[pltpu.VMEM((M1, 64 * TI), f32)],
    )(h1, pos1, pos1T, pos2, pos2flat, wa2, wv2, bvflat2, wvblk2, w2blk2,
      penblk2, b2tile2)
    h2 = h2.reshape(B, M2, 128)

    x3 = jnp.concatenate([h2, pos2], axis=-1).reshape(B * M2, 131)
    out = pl.pallas_call(
        _head_kernel,
        grid=(1,),
        in_specs=[
            pl.BlockSpec((B * M2, 131), lambda i: (0, 0)),
            pl.BlockSpec((131, 256), lambda i: (0, 0)),
            pl.BlockSpec((1, 256), lambda i: (0, 0)),
            pl.BlockSpec((256, 512), lambda i: (0, 0)),
            pl.BlockSpec((1, 512), lambda i: (0, 0)),
            pl.BlockSpec((512, 256), lambda i: (0, 0)),
            pl.BlockSpec((1, 256), lambda i: (0, 0)),
            pl.BlockSpec((256, 40), lambda i: (0, 0)),
            pl.BlockSpec((1, 40), lambda i: (0, 0)),
        ],
        out_specs=pl.BlockSpec((B, 40), lambda i: (0, 0)),
        out_shape=jax.ShapeDtypeStruct((B, 40), f32),
    )(x3, W3_1, b3_1.reshape(1, 256), W3_2, b3_2.reshape(1, 512),
      Wh1, bh1.reshape(1, 256), Wh2, bh2.reshape(1, 40))
    return out
